# baseline (device time: 24896 ns/iter reference)
import jax
import jax.numpy as jnp
from jax import lax
from jax.experimental import pallas as pl
from jax.experimental.pallas import tpu as pltpu

N_DEV = 4
NC = 4


def kernel(x):
    m_per, n = x.shape
    h = m_per // 2
    q = m_per // 4
    s = q // NC

    def body(x_ref, out_ref, acc_a, acc_b,
             recv_a1, recv_b1, recv_a2, recv_b2, send_sems, recv_sems):
        me = lax.axis_index("i")
        kx = me // 2
        ky = kx ^ (me & 1)
        nbr_y = me ^ 1
        nbr_x = 3 - me

        barrier_sem = pltpu.get_barrier_semaphore()
        for nbr in (nbr_x, nbr_y):
            pl.semaphore_signal(
                barrier_sem, inc=1,
                device_id=(nbr,), device_id_type=pl.DeviceIdType.MESH,
            )
        pl.semaphore_wait(barrier_sem, 2)

        def xchg(src, dst, sem_idx, nbr):
            return pltpu.make_async_remote_copy(
                src_ref=src, dst_ref=dst,
                send_sem=send_sems.at[sem_idx], recv_sem=recv_sems.at[sem_idx],
                device_id=(nbr,), device_id_type=pl.DeviceIdType.MESH,
            )

        a_off = kx * q
        b_off = h + ky * q
        a_send = (1 - kx) * q
        b_send = h + (1 - ky) * q

        a1 = [xchg(x_ref.at[pl.ds(a_send + c * s, s)], recv_a1.at[pl.ds(c * s, s)],
                   0 * NC + c, nbr_x) for c in range(NC)]
        b1 = [xchg(x_ref.at[pl.ds(b_send + c * s, s)], recv_b1.at[pl.ds(c * s, s)],
                   1 * NC + c, nbr_y) for c in range(NC)]
        for c in range(NC):
            a1[c].start()
            b1[c].start()

        a2 = [xchg(acc_a.at[pl.ds(c * s, s)], recv_a2.at[pl.ds(c * s, s)],
                   2 * NC + c, nbr_y) for c in range(NC)]
        b2 = [xchg(acc_b.at[pl.ds(c * s, s)], recv_b2.at[pl.ds(c * s, s)],
                   3 * NC + c, nbr_x) for c in range(NC)]
        a3 = [xchg(out_ref.at[pl.ds(a_off + c * s, s)],
                   out_ref.at[pl.ds(a_off + c * s, s)], 4 * NC + c, nbr_x)
              for c in range(NC)]
        b3 = [xchg(out_ref.at[pl.ds(b_off + c * s, s)],
                   out_ref.at[pl.ds(b_off + c * s, s)], 5 * NC + c, nbr_y)
              for c in range(NC)]

        for c in range(NC):
            a1[c].wait()
            acc_a[pl.ds(c * s, s), :] = (
                x_ref[pl.ds(a_off + c * s, s), :] + recv_a1[pl.ds(c * s, s), :]
            )
            a2[c].start()
            b1[c].wait()
            acc_b[pl.ds(c * s, s), :] = (
                x_ref[pl.ds(b_off + c * s, s), :] + recv_b1[pl.ds(c * s, s), :]
            )
            b2[c].start()

        for c in range(NC):
            a2[c].wait()
            out_ref[pl.ds(a_off + c * s, s), :] = (
                acc_a[pl.ds(c * s, s), :] + recv_a2[pl.ds(c * s, s), :]
            )
            a3[c].start()
            b2[c].wait()
            out_ref[pl.ds(b_off + c * s, s), :] = (
                acc_b[pl.ds(c * s, s), :] + recv_b2[pl.ds(c * s, s), :]
            )
            b3[c].start()

        for c in range(NC):
            a3[c].wait()
            b3[c].wait()

    return pl.pallas_call(
        body,
        out_shape=jax.ShapeDtypeStruct((m_per, n), x.dtype),
        in_specs=[pl.BlockSpec(memory_space=pltpu.VMEM)],
        out_specs=pl.BlockSpec(memory_space=pltpu.VMEM),
        scratch_shapes=[
            pltpu.VMEM((q, n), x.dtype),
            pltpu.VMEM((q, n), x.dtype),
            pltpu.VMEM((q, n), x.dtype),
            pltpu.VMEM((q, n), x.dtype),
            pltpu.VMEM((q, n), x.dtype),
            pltpu.VMEM((q, n), x.dtype),
            pltpu.SemaphoreType.DMA((6 * NC,)),
            pltpu.SemaphoreType.DMA((6 * NC,)),
        ],
        compiler_params=pltpu.CompilerParams(collective_id=0),
    )(x)


# device time: 24714 ns/iter; 1.0074x vs baseline; 1.0074x over previous
import jax
import jax.numpy as jnp
from jax import lax
from jax.experimental import pallas as pl
from jax.experimental.pallas import tpu as pltpu

N_DEV = 4
NC = 2


def kernel(x):
    m_per, n = x.shape
    h = m_per // 2
    q = m_per // 4
    s = q // NC

    def body(x_ref, out_ref, acc_a, acc_b,
             recv_a1, recv_b1, recv_a2, recv_b2, send_sems, recv_sems):
        me = lax.axis_index("i")
        kx = me // 2
        ky = kx ^ (me & 1)
        nbr_y = me ^ 1
        nbr_x = 3 - me

        barrier_sem = pltpu.get_barrier_semaphore()
        for nbr in (nbr_x, nbr_y):
            pl.semaphore_signal(
                barrier_sem, inc=1,
                device_id=(nbr,), device_id_type=pl.DeviceIdType.MESH,
            )
        pl.semaphore_wait(barrier_sem, 2)

        def xchg(src, dst, sem_idx, nbr):
            return pltpu.make_async_remote_copy(
                src_ref=src, dst_ref=dst,
                send_sem=send_sems.at[sem_idx], recv_sem=recv_sems.at[sem_idx],
                device_id=(nbr,), device_id_type=pl.DeviceIdType.MESH,
            )

        a_off = kx * q
        b_off = h + ky * q
        a_send = (1 - kx) * q
        b_send = h + (1 - ky) * q

        a1 = [xchg(x_ref.at[pl.ds(a_send + c * s, s)], recv_a1.at[pl.ds(c * s, s)],
                   0 * NC + c, nbr_x) for c in range(NC)]
        b1 = [xchg(x_ref.at[pl.ds(b_send + c * s, s)], recv_b1.at[pl.ds(c * s, s)],
                   1 * NC + c, nbr_y) for c in range(NC)]
        for c in range(NC):
            a1[c].start()
            b1[c].start()

        a2 = [xchg(acc_a.at[pl.ds(c * s, s)], recv_a2.at[pl.ds(c * s, s)],
                   2 * NC + c, nbr_y) for c in range(NC)]
        b2 = [xchg(acc_b.at[pl.ds(c * s, s)], recv_b2.at[pl.ds(c * s, s)],
                   3 * NC + c, nbr_x) for c in range(NC)]
        a3 = [xchg(out_ref.at[pl.ds(a_off + c * s, s)],
                   out_ref.at[pl.ds(a_off + c * s, s)], 4 * NC + c, nbr_x)
              for c in range(NC)]
        b3 = [xchg(out_ref.at[pl.ds(b_off + c * s, s)],
                   out_ref.at[pl.ds(b_off + c * s, s)], 5 * NC + c, nbr_y)
              for c in range(NC)]

        for c in range(NC):
            a1[c].wait()
            acc_a[pl.ds(c * s, s), :] = recv_a1[pl.ds(c * s, s), :]
            a2[c].start()
            b1[c].wait()
            acc_b[pl.ds(c * s, s), :] = recv_b1[pl.ds(c * s, s), :]
            b2[c].start()

        for c in range(NC):
            a2[c].wait()
            out_ref[pl.ds(a_off + c * s, s), :] = recv_a2[pl.ds(c * s, s), :]
            a3[c].start()
            b2[c].wait()
            out_ref[pl.ds(b_off + c * s, s), :] = recv_b2[pl.ds(c * s, s), :]
            b3[c].start()

        for c in range(NC):
            a3[c].wait()
            b3[c].wait()

    return pl.pallas_call(
        body,
        out_shape=jax.ShapeDtypeStruct((m_per, n), x.dtype),
        in_specs=[pl.BlockSpec(memory_space=pltpu.VMEM)],
        out_specs=pl.BlockSpec(memory_space=pltpu.VMEM),
        scratch_shapes=[
            pltpu.VMEM((q, n), x.dtype),
            pltpu.VMEM((q, n), x.dtype),
            pltpu.VMEM((q, n), x.dtype),
            pltpu.VMEM((q, n), x.dtype),
            pltpu.VMEM((q, n), x.dtype),
            pltpu.VMEM((q, n), x.dtype),
            pltpu.SemaphoreType.DMA((6 * NC,)),
            pltpu.SemaphoreType.DMA((6 * NC,)),
        ],
        compiler_params=pltpu.CompilerParams(collective_id=0),
    )(x)


# device time: 23605 ns/iter; 1.0547x vs baseline; 1.0470x over previous
import jax
import jax.numpy as jnp
from jax import lax
from jax.experimental import pallas as pl
from jax.experimental.pallas import tpu as pltpu

N_DEV = 4
NC = 2


def kernel(x):
    m_per, n = x.shape
    h = m_per // 2
    q = m_per // 4
    s = q // NC

    def body(x_ref, out_ref, acc_a, acc_b,
             recv_a1, recv_b1, recv_a2, recv_b2, send_sems, recv_sems):
        me = lax.axis_index("i")
        kx = me // 2
        ky = kx ^ (me & 1)
        nbr_y = me ^ 1
        nbr_x = 3 - me

        barrier_sem = pltpu.get_barrier_semaphore()
        for nbr in (nbr_x, nbr_y):
            pl.semaphore_signal(
                barrier_sem, inc=1,
                device_id=(nbr,), device_id_type=pl.DeviceIdType.MESH,
            )
        pl.semaphore_wait(barrier_sem, 2)

        def xchg(src, dst, sem_idx, nbr):
            return pltpu.make_async_remote_copy(
                src_ref=src, dst_ref=dst,
                send_sem=send_sems.at[sem_idx], recv_sem=recv_sems.at[sem_idx],
                device_id=(nbr,), device_id_type=pl.DeviceIdType.MESH,
            )

        a_off = kx * q
        b_off = h + ky * q
        a_send = (1 - kx) * q
        b_send = h + (1 - ky) * q

        a1 = [xchg(x_ref.at[pl.ds(a_send + c * s, s)], recv_a1.at[pl.ds(c * s, s)],
                   0 * NC + c, nbr_x) for c in range(NC)]
        b1 = [xchg(x_ref.at[pl.ds(b_send + c * s, s)], recv_b1.at[pl.ds(c * s, s)],
                   1 * NC + c, nbr_y) for c in range(NC)]
        for c in range(NC):
            a1[c].start()
            b1[c].start()

        a2 = [xchg(acc_a.at[pl.ds(c * s, s)], recv_a2.at[pl.ds(c * s, s)],
                   2 * NC + c, nbr_y) for c in range(NC)]
        b2 = [xchg(acc_b.at[pl.ds(c * s, s)], recv_b2.at[pl.ds(c * s, s)],
                   3 * NC + c, nbr_x) for c in range(NC)]
        a3 = [xchg(out_ref.at[pl.ds(a_off + c * s, s)],
                   out_ref.at[pl.ds(a_off + c * s, s)], 4 * NC + c, nbr_x)
              for c in range(NC)]
        b3 = [xchg(out_ref.at[pl.ds(b_off + c * s, s)],
                   out_ref.at[pl.ds(b_off + c * s, s)], 5 * NC + c, nbr_y)
              for c in range(NC)]

        for c in range(NC):
            a1[c].wait()
            acc_a[pl.ds(c * s, s), :] = (
                x_ref[pl.ds(a_off + c * s, s), :] + recv_a1[pl.ds(c * s, s), :]
            )
            a2[c].start()
            b1[c].wait()
            acc_b[pl.ds(c * s, s), :] = (
                x_ref[pl.ds(b_off + c * s, s), :] + recv_b1[pl.ds(c * s, s), :]
            )
            b2[c].start()

        for c in range(NC):
            a2[c].wait()
            out_ref[pl.ds(a_off + c * s, s), :] = (
                acc_a[pl.ds(c * s, s), :] + recv_a2[pl.ds(c * s, s), :]
            )
            a3[c].start()
            b2[c].wait()
            out_ref[pl.ds(b_off + c * s, s), :] = (
                acc_b[pl.ds(c * s, s), :] + recv_b2[pl.ds(c * s, s), :]
            )
            b3[c].start()

        for c in range(NC):
            a3[c].wait()
            b3[c].wait()

    return pl.pallas_call(
        body,
        out_shape=jax.ShapeDtypeStruct((m_per, n), x.dtype),
        in_specs=[pl.BlockSpec(memory_space=pltpu.VMEM)],
        out_specs=pl.BlockSpec(memory_space=pltpu.VMEM),
        scratch_shapes=[
            pltpu.VMEM((q, n), x.dtype),
            pltpu.VMEM((q, n), x.dtype),
            pltpu.VMEM((q, n), x.dtype),
            pltpu.VMEM((q, n), x.dtype),
            pltpu.VMEM((q, n), x.dtype),
            pltpu.VMEM((q, n), x.dtype),
            pltpu.SemaphoreType.DMA((6 * NC,)),
            pltpu.SemaphoreType.DMA((6 * NC,)),
        ],
        compiler_params=pltpu.CompilerParams(collective_id=0),
    )(x)
